# baseline (speedup 1.0000x reference)
import functools

import jax
import jax.numpy as jnp
from jax import lax
from jax.experimental import pallas as pl
from jax.experimental.pallas import tpu as pltpu
from jax.experimental.pallas import tpu_sc as plsc

VOCAB = 1000
B, T = 1024, 200
NW = 32
C = 50
NCHUNK = (B * T) // (NW * C)
NBUF = 2


def _body(table_hbm, idx_hbm, out_hbm, idx_v, buf0, buf1,
          gsem0, gsem1, osem0, osem1):
    bufs = (buf0, buf1)
    gsems = (gsem0, gsem1)
    osems = (osem0, osem1)
    wid = lax.axis_index("s") * 2 + lax.axis_index("c")

    pltpu.sync_copy(idx_hbm.at[wid], idx_v)

    def gather_start(j, b):
        pltpu.async_copy(table_hbm.at[idx_v.at[j]], bufs[b], gsems[b])

    def gather_wait(j, b):
        pltpu.make_async_copy(table_hbm.at[idx_v.at[j]], bufs[b],
                              gsems[b]).wait()

    def write_start(j, b):
        pltpu.async_copy(bufs[b], out_hbm.at[wid, j], osems[b])

    def write_wait(j, b):
        pltpu.make_async_copy(bufs[b], out_hbm.at[wid, j], osems[b]).wait()

    for b in range(NBUF):
        gather_start(b, b)

    @pl.loop(0, (NCHUNK - NBUF) // NBUF)
    def _(k):
        for b in range(NBUF):
            j = k * NBUF + b
            gather_wait(j, b)
            write_start(j, b)
            write_wait(j, b)
            gather_start(j + NBUF, b)

    for b in range(NBUF):
        j = NCHUNK - NBUF + b
        gather_wait(j, b)
        write_start(j, b)
        write_wait(j, b)


def kernel(inps, table):
    idx = inps.reshape(NW, NCHUNK, C)
    mesh = plsc.VectorSubcoreMesh(core_axis_name="c", subcore_axis_name="s")
    run = pl.kernel(
        _body,
        out_type=jax.ShapeDtypeStruct((NW, NCHUNK, C, VOCAB), jnp.float32),
        mesh=mesh,
        scratch_types=[
            pltpu.VMEM((NCHUNK, C), jnp.int32),
            pltpu.VMEM((C, VOCAB), jnp.float32),
            pltpu.VMEM((C, VOCAB), jnp.float32),
            pltpu.SemaphoreType.DMA,
            pltpu.SemaphoreType.DMA,
            pltpu.SemaphoreType.DMA,
            pltpu.SemaphoreType.DMA,
        ],
        compiler_params=pltpu.CompilerParams(use_tc_tiling_on_sc=False),
    )
    out = run(table, idx)
    return out.reshape(B, T, VOCAB)


# speedup vs baseline: 1.0013x; 1.0013x over previous
import functools

import jax
import jax.numpy as jnp
from jax import lax
from jax.experimental import pallas as pl
from jax.experimental.pallas import tpu as pltpu
from jax.experimental.pallas import tpu_sc as plsc

VOCAB = 1000
B, T = 1024, 200
NW = 32
C = 50
NCHUNK = (B * T) // (NW * C)
NBUF = 2


def _body(table_hbm, idx_hbm, out_hbm, idx_v, buf0, buf1,
          gsem0, gsem1, osem0, osem1):
    bufs = (buf0, buf1)
    gsems = (gsem0, gsem1)
    osems = (osem0, osem1)
    wid = lax.axis_index("s") * 2 + lax.axis_index("c")

    pltpu.sync_copy(idx_hbm.at[wid], idx_v)

    def out_slice(j):
        b_row = wid * (6400 // T) + j // (T // C)
        t0 = (j % (T // C)) * C
        return out_hbm.at[b_row, pl.ds(t0, C)]

    def gather_start(j, b):
        pltpu.async_copy(table_hbm.at[idx_v.at[j]], bufs[b], gsems[b])

    def gather_wait(j, b):
        pltpu.make_async_copy(table_hbm.at[idx_v.at[j]], bufs[b],
                              gsems[b]).wait()

    def write_start(j, b):
        pltpu.async_copy(bufs[b], out_slice(j), osems[b])

    def write_wait(j, b):
        pltpu.make_async_copy(bufs[b], out_slice(j), osems[b]).wait()

    for b in range(NBUF):
        gather_start(b, b)

    @pl.loop(0, (NCHUNK - NBUF) // NBUF)
    def _(k):
        for b in range(NBUF):
            j = k * NBUF + b
            gather_wait(j, b)
            write_start(j, b)
            write_wait(j, b)
            gather_start(j + NBUF, b)

    for b in range(NBUF):
        j = NCHUNK - NBUF + b
        gather_wait(j, b)
        write_start(j, b)
        write_wait(j, b)


def kernel(inps, table):
    idx = inps.reshape(NW, NCHUNK, C)
    mesh = plsc.VectorSubcoreMesh(core_axis_name="c", subcore_axis_name="s")
    run = pl.kernel(
        _body,
        out_type=jax.ShapeDtypeStruct((B, T, VOCAB), jnp.float32),
        mesh=mesh,
        scratch_types=[
            pltpu.VMEM((NCHUNK, C), jnp.int32),
            pltpu.VMEM((C, VOCAB), jnp.float32),
            pltpu.VMEM((C, VOCAB), jnp.float32),
            pltpu.SemaphoreType.DMA,
            pltpu.SemaphoreType.DMA,
            pltpu.SemaphoreType.DMA,
            pltpu.SemaphoreType.DMA,
        ],
        compiler_params=pltpu.CompilerParams(use_tc_tiling_on_sc=False),
    )
    return run(table, idx)
